# baseline (speedup 1.0000x reference)
import functools

import jax
import jax.numpy as jnp
from jax.experimental import pallas as pl
from jax.experimental.pallas import tpu as pltpu

KB = 2048
NLANES = 128
BIG_F = 1.0e37
BIG_I = 2 ** 30

DOT_PRECISION = jax.lax.Precision.DEFAULT


def _knn_kernel(q_ref, k_ref, score_ref, idx_ref, rv_ref, ri_ref, *, nsteps, nkeys):
    pid = pl.program_id(0)

    @pl.when(pid == 0)
    def _init():
        rv_ref[...] = jnp.full_like(rv_ref, -3.0e38)
        ri_ref[...] = jnp.zeros_like(ri_ref)

    kb = k_ref[...]
    base = pid * KB
    q = q_ref[...]
    dot = jax.lax.dot_general(
        q, kb, (((1,), (1,)), ((), ())),
        preferred_element_type=jnp.float32,
        precision=DOT_PRECISION,
    )

    hk = 0.5 * jnp.sum(kb * kb, axis=1)
    col_id = jax.lax.broadcasted_iota(jnp.int32, (1, KB), 1) + base
    hk = jnp.where(col_id < nkeys, hk.reshape(1, KB), BIG_F)

    s = dot - hk

    lane = jax.lax.broadcasted_iota(jnp.int32, (1, NLANES), 1)
    for c in range(KB // NLANES):
        chunk = s[:, c * NLANES:(c + 1) * NLANES]
        ci = lane + (base + c * NLANES)
        rv = rv_ref[...]
        upd = chunk > rv
        rv_ref[...] = jnp.where(upd, chunk, rv)
        ri_ref[...] = jnp.where(upd, jnp.broadcast_to(ci, upd.shape), ri_ref[...])

    @pl.when(pid == nsteps - 1)
    def _emit():
        rvf = rv_ref[...]
        rif = ri_ref[...]
        m = jnp.max(rvf, axis=1, keepdims=True)
        cand = jnp.where(rvf == m, rif, BIG_I)
        best = jnp.min(cand, axis=1, keepdims=True)
        q_sq = jnp.sum(q * q, axis=1, keepdims=True)
        d2 = jnp.maximum(q_sq - 2.0 * m, 0.0)
        score_ref[...] = jnp.sqrt(d2)
        idx_ref[...] = best


def kernel(queries, keys):
    nq, d = queries.shape
    nkeys = keys.shape[0]
    nsteps = pl.cdiv(nkeys, KB)

    score, idx = pl.pallas_call(
        functools.partial(_knn_kernel, nsteps=nsteps, nkeys=nkeys),
        grid=(nsteps,),
        in_specs=[
            pl.BlockSpec((nq, d), lambda i: (0, 0)),
            pl.BlockSpec((KB, d), lambda i: (i, 0)),
        ],
        out_specs=[
            pl.BlockSpec((nq, 1), lambda i: (0, 0)),
            pl.BlockSpec((nq, 1), lambda i: (0, 0)),
        ],
        out_shape=[
            jax.ShapeDtypeStruct((nq, 1), jnp.float32),
            jax.ShapeDtypeStruct((nq, 1), jnp.int32),
        ],
        scratch_shapes=[
            pltpu.VMEM((nq, NLANES), jnp.float32),
            pltpu.VMEM((nq, NLANES), jnp.int32),
        ],
    )(queries, keys)

    return score.reshape(nq), idx


# speedup vs baseline: 1.0103x; 1.0103x over previous
import functools

import jax
import jax.numpy as jnp
from jax.experimental import pallas as pl
from jax.experimental.pallas import tpu as pltpu

KB = 4096
NLANES = 128
BIG_F = 1.0e37
BIG_I = 2 ** 30

DOT_PRECISION = jax.lax.Precision.DEFAULT


def _knn_kernel(q_ref, k_ref, score_ref, idx_ref, rv_ref, ri_ref, *, nsteps, nkeys):
    pid = pl.program_id(0)

    @pl.when(pid == 0)
    def _init():
        rv_ref[...] = jnp.full_like(rv_ref, -3.0e38)
        ri_ref[...] = jnp.zeros_like(ri_ref)

    kb = k_ref[...]
    base = pid * KB
    q = q_ref[...]
    dot = jax.lax.dot_general(
        q, kb, (((1,), (1,)), ((), ())),
        preferred_element_type=jnp.float32,
        precision=DOT_PRECISION,
    )

    hk = 0.5 * jnp.sum(kb * kb, axis=1)
    col_id = jax.lax.broadcasted_iota(jnp.int32, (1, KB), 1) + base
    hk = jnp.where(col_id < nkeys, hk.reshape(1, KB), BIG_F)

    lane = jax.lax.broadcasted_iota(jnp.int32, (1, NLANES), 1)
    for c in range(KB // NLANES):
        sl = slice(c * NLANES, (c + 1) * NLANES)
        chunk = dot[:, sl] - hk[:, sl]
        ci = lane + (base + c * NLANES)
        rv = rv_ref[...]
        upd = chunk > rv
        rv_ref[...] = jnp.where(upd, chunk, rv)
        ri_ref[...] = jnp.where(upd, jnp.broadcast_to(ci, upd.shape), ri_ref[...])

    @pl.when(pid == nsteps - 1)
    def _emit():
        rvf = rv_ref[...]
        rif = ri_ref[...]
        m = jnp.max(rvf, axis=1, keepdims=True)
        cand = jnp.where(rvf == m, rif, BIG_I)
        best = jnp.min(cand, axis=1, keepdims=True)
        q_sq = jnp.sum(q * q, axis=1, keepdims=True)
        d2 = jnp.maximum(q_sq - 2.0 * m, 0.0)
        score_ref[...] = jnp.sqrt(d2)
        idx_ref[...] = best


def kernel(queries, keys):
    nq, d = queries.shape
    nkeys = keys.shape[0]
    nsteps = pl.cdiv(nkeys, KB)

    score, idx = pl.pallas_call(
        functools.partial(_knn_kernel, nsteps=nsteps, nkeys=nkeys),
        grid=(nsteps,),
        in_specs=[
            pl.BlockSpec((nq, d), lambda i: (0, 0)),
            pl.BlockSpec((KB, d), lambda i: (i, 0)),
        ],
        out_specs=[
            pl.BlockSpec((nq, 1), lambda i: (0, 0)),
            pl.BlockSpec((nq, 1), lambda i: (0, 0)),
        ],
        out_shape=[
            jax.ShapeDtypeStruct((nq, 1), jnp.float32),
            jax.ShapeDtypeStruct((nq, 1), jnp.int32),
        ],
        scratch_shapes=[
            pltpu.VMEM((nq, NLANES), jnp.float32),
            pltpu.VMEM((nq, NLANES), jnp.int32),
        ],
    )(queries, keys)

    return score.reshape(nq), idx
